# baseline (speedup 1.0000x reference)
import functools

import jax
import jax.numpy as jnp
from jax import lax
from jax.experimental import pallas as pl
from jax.experimental.pallas import tpu as pltpu
from jax.experimental.pallas import tpu_sc as plsc

NUM_EMB = 1000000
DIM = 64

NC = 2
NS = 16
NW = NC * NS

WBR = 2000
CHUNK = 128


def _widen_block(t_ref, o_ref):
    o_ref[:, pl.ds(0, DIM)] = t_ref[...]
    o_ref[:, pl.ds(DIM, DIM)] = jnp.zeros((WBR, DIM), jnp.float32)


def _gather_body(nchunk, idx_hbm, wide_hbm, out_hbm, idx_v, rows_v,
                 gsem, wsem):
    wid = lax.axis_index("s") * NC + lax.axis_index("c")
    base = wid * nchunk

    pltpu.sync_copy(idx_hbm.at[pl.ds(base, nchunk)], idx_v)

    def gather(g, slot):
        return pltpu.async_copy(wide_hbm.at[idx_v.at[g]], rows_v.at[slot],
                                gsem)

    gather(0, 0)

    def loop_body(g, carry):
        slot = lax.rem(g, 2)

        @pl.when(g >= 1)
        def _():
            pltpu.make_async_copy(
                rows_v.at[slot], out_hbm.at[pl.ds(0, CHUNK)], wsem).wait()

        @pl.when(g + 1 < nchunk)
        def _():
            gather(g + 1, 1 - slot)

        pltpu.make_async_copy(wide_hbm.at[idx_v.at[g]], rows_v.at[slot],
                              gsem).wait()
        pltpu.async_copy(
            rows_v.at[slot],
            out_hbm.at[pl.ds((base + g) * CHUNK, CHUNK)], wsem)
        return carry

    lax.fori_loop(0, nchunk, loop_body, 0)

    pltpu.make_async_copy(
        rows_v.at[0], out_hbm.at[pl.ds(0, CHUNK)], wsem).wait()


@functools.partial(jax.jit, static_argnames=("nidx",))
def _run(x2d, table, nidx):
    wide = pl.pallas_call(
        _widen_block,
        grid=(NUM_EMB // WBR,),
        in_specs=[pl.BlockSpec((WBR, DIM), lambda i: (i, 0))],
        out_specs=pl.BlockSpec((WBR, 128), lambda i: (i, 0)),
        out_shape=jax.ShapeDtypeStruct((NUM_EMB, 128), jnp.float32),
    )(table)

    nchunk = nidx // (NW * CHUNK)
    mesh = plsc.VectorSubcoreMesh(core_axis_name="c", subcore_axis_name="s")
    out_wide = functools.partial(
        pl.kernel,
        out_type=jax.ShapeDtypeStruct((nidx, 128), jnp.float32),
        mesh=mesh,
        scratch_types=[
            pltpu.VMEM((nchunk, CHUNK), jnp.int32),
            pltpu.VMEM((2, CHUNK, 128), jnp.float32),
            pltpu.SemaphoreType.DMA,
            pltpu.SemaphoreType.DMA,
        ],
    )(functools.partial(_gather_body, nchunk))(x2d, wide)
    return out_wide


def kernel(x, table):
    nbatch, seq = x.shape
    nidx = nbatch * seq
    assert nidx % (NW * CHUNK) == 0
    x2d = x.reshape(nidx // CHUNK, CHUNK).astype(jnp.int32)
    out_wide = _run(x2d, table, nidx)
    return out_wide[:, :DIM].reshape(nbatch, seq, DIM)


# speedup vs baseline: 1.0744x; 1.0744x over previous
import functools

import jax
import jax.numpy as jnp
from jax import lax
from jax.experimental import pallas as pl
from jax.experimental.pallas import tpu as pltpu
from jax.experimental.pallas import tpu_sc as plsc

NUM_EMB = 1000000
DIM = 64

NC = 2
NS = 16
NW = NC * NS


def _gather_body(b_per_w, seq, idx_hbm, table_hbm, out_hbm, idx_v, rows_v,
                 gsem, wsem):
    wid = lax.axis_index("s") * NC + lax.axis_index("c")
    base = wid * b_per_w

    pltpu.sync_copy(idx_hbm.at[pl.ds(base, b_per_w)], idx_v)

    def gather(b, slot):
        return pltpu.async_copy(table_hbm.at[idx_v.at[b]], rows_v.at[slot],
                                gsem)

    gather(0, 0)

    def loop_body(b, carry):
        slot = lax.rem(b, 2)

        @pl.when(b >= 1)
        def _():
            pltpu.make_async_copy(rows_v.at[slot], out_hbm.at[base],
                                  wsem).wait()

        @pl.when(b + 1 < b_per_w)
        def _():
            gather(b + 1, 1 - slot)

        pltpu.make_async_copy(table_hbm.at[idx_v.at[b]], rows_v.at[slot],
                              gsem).wait()
        pltpu.async_copy(rows_v.at[slot], out_hbm.at[base + b], wsem)
        return carry

    lax.fori_loop(0, b_per_w, loop_body, 0)

    pltpu.make_async_copy(rows_v.at[0], out_hbm.at[base], wsem).wait()


@functools.partial(jax.jit, static_argnames=("nbatch", "seq"))
def _run(x, table, nbatch, seq):
    mesh = plsc.VectorSubcoreMesh(core_axis_name="c", subcore_axis_name="s")
    return functools.partial(
        pl.kernel,
        out_type=jax.ShapeDtypeStruct((nbatch, seq, DIM), jnp.float32),
        mesh=mesh,
        scratch_types=[
            pltpu.VMEM((nbatch // NW, seq), jnp.int32),
            pltpu.VMEM((2, seq, DIM), jnp.float32),
            pltpu.SemaphoreType.DMA,
            pltpu.SemaphoreType.DMA,
        ],
        compiler_params=pltpu.CompilerParams(use_tc_tiling_on_sc=False),
    )(functools.partial(_gather_body, nbatch // NW, seq))(x, table)


def kernel(x, table):
    nbatch, seq = x.shape
    assert nbatch % NW == 0
    return _run(x.astype(jnp.int32), table, nbatch, seq)


# speedup vs baseline: 1.1591x; 1.0788x over previous
import functools

import jax
import jax.numpy as jnp
from jax import lax
from jax.experimental import pallas as pl
from jax.experimental.pallas import tpu as pltpu
from jax.experimental.pallas import tpu_sc as plsc

NUM_EMB = 1000000
DIM = 64

NC = 2
NS = 16
NW = NC * NS

WBR = 8000


def _widen_block(t_ref, o_ref):
    o_ref[:, pl.ds(0, DIM)] = t_ref[...]
    o_ref[:, pl.ds(DIM, DIM)] = jnp.zeros((WBR, DIM), jnp.float32)


CHUNK = 128


def _gather_body(nchunk, idx_hbm, wide_hbm, out_hbm, idx_v, rows_v,
                 gsem, wsem):
    wid = lax.axis_index("s") * NC + lax.axis_index("c")
    base = wid * nchunk

    pltpu.sync_copy(idx_hbm.at[pl.ds(base, nchunk)], idx_v)

    def gather(g, slot):
        return pltpu.async_copy(wide_hbm.at[idx_v.at[g]], rows_v.at[slot],
                                gsem)

    gather(0, 0)

    def loop_body(g, carry):
        slot = lax.rem(g, 2)

        @pl.when(g >= 1)
        def _():
            pltpu.make_async_copy(
                rows_v.at[slot], out_hbm.at[pl.ds(0, CHUNK)], wsem).wait()

        @pl.when(g + 1 < nchunk)
        def _():
            gather(g + 1, 1 - slot)

        pltpu.make_async_copy(wide_hbm.at[idx_v.at[g]], rows_v.at[slot],
                              gsem).wait()
        pltpu.async_copy(
            rows_v.at[slot],
            out_hbm.at[pl.ds((base + g) * CHUNK, CHUNK)], wsem)
        return carry

    lax.fori_loop(0, nchunk, loop_body, 0)

    pltpu.make_async_copy(
        rows_v.at[0], out_hbm.at[pl.ds(0, CHUNK)], wsem).wait()


@functools.partial(jax.jit, static_argnames=("nidx",))
def _run(x2d, table, nidx):
    wide = pl.pallas_call(
        _widen_block,
        grid=(NUM_EMB // WBR,),
        in_specs=[pl.BlockSpec((WBR, DIM), lambda i: (i, 0))],
        out_specs=pl.BlockSpec((WBR, 128), lambda i: (i, 0)),
        out_shape=jax.ShapeDtypeStruct((NUM_EMB, 128), jnp.float32),
    )(table)

    nchunk = nidx // (NW * CHUNK)
    mesh = plsc.VectorSubcoreMesh(core_axis_name="c", subcore_axis_name="s")
    return functools.partial(
        pl.kernel,
        out_type=jax.ShapeDtypeStruct((nidx, 128), jnp.float32),
        mesh=mesh,
        scratch_types=[
            pltpu.VMEM((nchunk, CHUNK), jnp.int32),
            pltpu.VMEM((2, CHUNK, 128), jnp.float32),
            pltpu.SemaphoreType.DMA,
            pltpu.SemaphoreType.DMA,
        ],
    )(functools.partial(_gather_body, nchunk))(x2d, wide)


def kernel(x, table):
    nbatch, seq = x.shape
    nidx = nbatch * seq
    assert nidx % (NW * CHUNK) == 0
    x2d = x.reshape(nidx // CHUNK, CHUNK).astype(jnp.int32)
    out_wide = _run(x2d, table, nidx)
    return out_wide.reshape(nbatch, seq, 128)[:, :, :DIM]


# speedup vs baseline: 1.3100x; 1.1302x over previous
import functools

import jax
import jax.numpy as jnp
from jax import lax
from jax.experimental import pallas as pl
from jax.experimental.pallas import tpu as pltpu
from jax.experimental.pallas import tpu_sc as plsc

NUM_EMB = 1000000
DIM = 64

NC = 2
NS = 16
NW = NC * NS

WBR = 8000


def _widen_block(t_ref, o_ref):
    o_ref[:, pl.ds(0, DIM)] = t_ref[...]
    o_ref[:, pl.ds(DIM, DIM)] = jnp.zeros((WBR, DIM), jnp.float32)


CHUNK = 128


def _gather_body(nchunk, idx_hbm, wide_hbm, out_hbm, idx_v, rows_v,
                 gsem, wsem):
    wid = lax.axis_index("s") * NC + lax.axis_index("c")
    base = wid * nchunk

    pltpu.sync_copy(idx_hbm.at[pl.ds(base, nchunk)], idx_v)

    def gather(g, slot):
        return pltpu.async_copy(wide_hbm.at[idx_v.at[g]], rows_v.at[slot],
                                gsem)

    gather(0, 0)

    def loop_body(g, carry):
        slot = lax.rem(g, 2)

        @pl.when(g >= 1)
        def _():
            pltpu.make_async_copy(
                rows_v.at[slot], out_hbm.at[pl.ds(0, CHUNK)], wsem).wait()

        @pl.when(g + 1 < nchunk)
        def _():
            gather(g + 1, 1 - slot)

        pltpu.make_async_copy(wide_hbm.at[idx_v.at[g]], rows_v.at[slot],
                              gsem).wait()
        pltpu.async_copy(
            rows_v.at[slot],
            out_hbm.at[pl.ds((base + g) * CHUNK, CHUNK)], wsem)
        return carry

    lax.fori_loop(0, nchunk, loop_body, 0)

    pltpu.make_async_copy(
        rows_v.at[0], out_hbm.at[pl.ds(0, CHUNK)], wsem).wait()


@functools.partial(jax.jit, static_argnames=("nidx",))
def _run(x2d, table, nidx):
    wide = jnp.pad(table, ((0, 0), (0, 128 - DIM)))

    nchunk = nidx // (NW * CHUNK)
    mesh = plsc.VectorSubcoreMesh(core_axis_name="c", subcore_axis_name="s")
    return functools.partial(
        pl.kernel,
        out_type=jax.ShapeDtypeStruct((nidx, 128), jnp.float32),
        mesh=mesh,
        scratch_types=[
            pltpu.VMEM((nchunk, CHUNK), jnp.int32),
            pltpu.VMEM((2, CHUNK, 128), jnp.float32),
            pltpu.SemaphoreType.DMA,
            pltpu.SemaphoreType.DMA,
        ],
    )(functools.partial(_gather_body, nchunk))(x2d, wide)


def kernel(x, table):
    nbatch, seq = x.shape
    nidx = nbatch * seq
    assert nidx % (NW * CHUNK) == 0
    x2d = x.reshape(nidx // CHUNK, CHUNK).astype(jnp.int32)
    out_wide = _run(x2d, table, nidx)
    return out_wide.reshape(nbatch, seq, 128)[:, :, :DIM]
